# baseline (speedup 1.0000x reference)
import functools

import jax
import jax.numpy as jnp
from jax import lax
from jax.experimental import pallas as pl
from jax.experimental.pallas import tpu as pltpu
from jax.experimental.pallas import tpu_sc as plsc

T = 2048
H = 1024
F = 2048
E = 8
BS = 128
BS_LOG = 7
NB = 40
NB2 = 64
NP = NB * BS
LANES = 128
NCK = 4
NC, NS = 2, 16
NW = NC * NS
TPW = T // NW


def _route_body(x_ref, g_ref, pos1_ref, pos2_ref, w1_ref, w2_ref,
                eob_ref, nba_ref, isf_ref, pfx_ref, slot_ref, haspf_ref):
    x = x_ref[...]
    g = g_ref[...]
    logits = lax.dot_general(x, g, (((1,), (1,)), ((), ())),
                             preferred_element_type=jnp.float32)
    lane = lax.broadcasted_iota(jnp.int32, (T, LANES), 1)
    logits = jnp.where(lane < E, logits, jnp.float32(-1e30))
    m = jnp.max(logits, axis=1, keepdims=True)
    p = jnp.exp(logits - m)
    p = jnp.where(lane < E, p, 0.0)
    probs = p / jnp.sum(p, axis=1, keepdims=True)
    p1 = jnp.max(probs, axis=1, keepdims=True)
    e1 = jnp.min(jnp.where(probs == p1, lane, LANES), axis=1, keepdims=True)
    probs_m = jnp.where(lane == e1, -1.0, probs)
    p2 = jnp.max(probs_m, axis=1, keepdims=True)
    e2 = jnp.min(jnp.where(probs_m == p2, lane, LANES), axis=1, keepdims=True)
    tot = p1 + p2
    w1_ref[...] = p1 / tot
    w2_ref[...] = p2 / tot
    cnt = (lane == e1).astype(jnp.int32) + (lane == e2).astype(jnp.int32)
    inc = cnt
    sdist = 1
    while sdist < T:
        inc = inc + jnp.concatenate(
            [jnp.zeros((sdist, LANES), jnp.int32), inc[:T - sdist]], axis=0)
        sdist *= 2
    exc = inc - cnt
    n = inc[T - 1:T, :]
    nb = lax.shift_right_logical(n + (BS - 1), BS_LOG)
    spad = lax.shift_left(nb, BS_LOG)
    binc = nb
    ainc = spad
    for sh in (1, 2, 4):
        z = jnp.zeros((1, sh), jnp.int32)
        binc = binc + jnp.concatenate([z, binc[:, :LANES - sh]], axis=1)
        ainc = ainc + jnp.concatenate([z, ainc[:, :LANES - sh]], axis=1)
    aexc = ainc - spad
    posb = exc + aexc
    pos1_ref[...] = jnp.sum(jnp.where(lane == e1, posb, 0), axis=1,
                            keepdims=True)
    pos2_ref[...] = jnp.sum(jnp.where(lane == e2, posb, 0), axis=1,
                            keepdims=True)
    lane1 = lax.broadcasted_iota(jnp.int32, (1, LANES), 1)
    lastact = jnp.max(jnp.where((n > 0) & (lane1 < E), lane1, 0), axis=1,
                      keepdims=True)
    biota = lax.broadcasted_iota(jnp.int32, (NB2, LANES), 0)
    laneb = lax.broadcasted_iota(jnp.int32, (NB2, LANES), 1)
    ge = (biota >= binc) & (laneb < E)
    cntge = jnp.sum(ge.astype(jnp.int32), axis=1, keepdims=True)
    eobv = jnp.minimum(cntge, lastact)
    eob_ref[...] = eobv
    nba_ref[...] = jnp.sum(jnp.where(lane1 < E, nb, 0), axis=1, keepdims=True)
    prev = jnp.concatenate(
        [jnp.full((1, 1), -1, jnp.int32), eobv[:NB2 - 1]], axis=0)
    isf_ref[...] = (eobv != prev).astype(jnp.int32)
    activeb = n > 0
    cand = jnp.where(activeb & (laneb > eobv) & (laneb < E), laneb, LANES)
    nxt = jnp.min(cand, axis=1, keepdims=True)
    haspf_ref[...] = (nxt < E).astype(jnp.int32)
    pfx_ref[...] = jnp.where(nxt < E, nxt, eobv)
    ordv = jnp.sum(jnp.where(activeb & (laneb <= eobv) & (laneb < E), 1, 0),
                   axis=1, keepdims=True)
    slot_ref[...] = jnp.bitwise_and(ordv - 1, 1)


def _route(flat, gwp):
    return pl.pallas_call(
        _route_body,
        out_shape=[
            jax.ShapeDtypeStruct((T, 1), jnp.int32),
            jax.ShapeDtypeStruct((T, 1), jnp.int32),
            jax.ShapeDtypeStruct((T, 1), jnp.float32),
            jax.ShapeDtypeStruct((T, 1), jnp.float32),
            jax.ShapeDtypeStruct((NB2, 1), jnp.int32),
            jax.ShapeDtypeStruct((1, 1), jnp.int32),
            jax.ShapeDtypeStruct((NB2, 1), jnp.int32),
            jax.ShapeDtypeStruct((NB2, 1), jnp.int32),
            jax.ShapeDtypeStruct((NB2, 1), jnp.int32),
            jax.ShapeDtypeStruct((NB2, 1), jnp.int32),
        ],
    )(flat, gwp)


def _dispatch(flat, pos1, pos2):
    mesh = plsc.VectorSubcoreMesh(core_axis_name="c", subcore_axis_name="s")
    CH = 32

    @functools.partial(
        pl.kernel,
        mesh=mesh,
        out_type=jax.ShapeDtypeStruct((NP, H), jnp.float32),
        scratch_types=[
            pltpu.VMEM((CH,), jnp.int32),
            pltpu.VMEM((CH,), jnp.int32),
            pltpu.VMEM((CH, H), jnp.float32),
            pltpu.SemaphoreType.DMA,
        ],
    )
    def k(flat_hbm, pos1_hbm, pos2_hbm, xs_hbm, idx1_v, idx2_v, buf_v, sem):
        wid = lax.axis_index("s") * NC + lax.axis_index("c")
        for sub in range(TPW // CH):
            base = wid * TPW + sub * CH
            pltpu.sync_copy(pos1_hbm.at[pl.ds(base, CH)], idx1_v)
            pltpu.sync_copy(pos2_hbm.at[pl.ds(base, CH)], idx2_v)
            pltpu.sync_copy(flat_hbm.at[pl.ds(base, CH)], buf_v)
            c1 = pltpu.async_copy(buf_v, xs_hbm.at[idx1_v], sem)
            c2 = pltpu.async_copy(buf_v, xs_hbm.at[idx2_v], sem)
            c1.wait()
            c2.wait()

    return k(flat, pos1, pos2)


def _gmm_body(eob_ref, nba_ref, isf_ref, pfx_ref, slot_ref, haspf_ref,
              x_ref, w1_any, b1_ref, w2_any, b2_ref, o_ref,
              w1buf, w2buf, sem):
    b = pl.program_id(0)
    s = slot_ref[b]

    def _start_weight_copy(e, slot):
        for c in range(NCK):
            pltpu.make_async_copy(
                w1_any.at[e, pl.ds(c * (F // NCK), F // NCK)],
                w1buf.at[slot, pl.ds(c * (F // NCK), F // NCK)],
                sem.at[slot]).start()
            pltpu.make_async_copy(
                w2_any.at[e, pl.ds(c * (H // NCK), H // NCK)],
                w2buf.at[slot, pl.ds(c * (H // NCK), H // NCK)],
                sem.at[slot]).start()

    @pl.when(b == 0)
    def _():
        _start_weight_copy(eob_ref[0], 0)

    @pl.when((isf_ref[b] == 1) & (haspf_ref[b] == 1))
    def _():
        _start_weight_copy(pfx_ref[b], 1 - s)

    @pl.when(isf_ref[b] == 1)
    def _():
        for c in range(NCK):
            pltpu.make_async_copy(
                w1_any.at[0, pl.ds(0, F // NCK)],
                w1buf.at[s, pl.ds(0, F // NCK)], sem.at[s]).wait()
            pltpu.make_async_copy(
                w2_any.at[0, pl.ds(0, H // NCK)],
                w2buf.at[s, pl.ds(0, H // NCK)], sem.at[s]).wait()

    @pl.when(b < nba_ref[0])
    def _():
        x = x_ref[...]
        h = lax.dot_general(x, w1buf[s], (((1,), (1,)), ((), ())),
                            preferred_element_type=jnp.float32)
        h = jnp.maximum(h + b1_ref[...], 0.0)
        y = lax.dot_general(h, w2buf[s], (((1,), (1,)), ((), ())),
                            preferred_element_type=jnp.float32)
        o_ref[...] = y + b2_ref[...]


def _gmm(xs, W1, b1, W2, b2, eob, nba, isf, pfx, slot, haspf):
    grid_spec = pltpu.PrefetchScalarGridSpec(
        num_scalar_prefetch=6,
        grid=(NB,),
        in_specs=[
            pl.BlockSpec((BS, H), lambda b, *sp: (b, 0)),
            pl.BlockSpec(memory_space=pl.ANY),
            pl.BlockSpec((None, 1, F), lambda b, *sp: (sp[0][b], 0, 0)),
            pl.BlockSpec(memory_space=pl.ANY),
            pl.BlockSpec((None, 1, H), lambda b, *sp: (sp[0][b], 0, 0)),
        ],
        out_specs=pl.BlockSpec((BS, H), lambda b, *sp: (b, 0)),
        scratch_shapes=[
            pltpu.VMEM((2, F, H), jnp.float32),
            pltpu.VMEM((2, H, F), jnp.float32),
            pltpu.SemaphoreType.DMA((2,)),
        ],
    )
    return pl.pallas_call(
        _gmm_body,
        grid_spec=grid_spec,
        out_shape=jax.ShapeDtypeStruct((NP, H), jnp.float32),
        compiler_params=pltpu.CompilerParams(
            dimension_semantics=("arbitrary",)),
    )(eob, nba, isf, pfx, slot, haspf,
      xs, W1, b1.reshape(E, 1, F), W2, b2.reshape(E, 1, H))


def _combine(ys, pos1, pos2, w1, w2):
    mesh = plsc.VectorSubcoreMesh(core_axis_name="c", subcore_axis_name="s")
    CH = 16

    @functools.partial(
        pl.kernel,
        mesh=mesh,
        out_type=jax.ShapeDtypeStruct((T, H), jnp.float32),
        scratch_types=[
            pltpu.VMEM((CH,), jnp.int32),
            pltpu.VMEM((CH,), jnp.int32),
            pltpu.VMEM((CH,), jnp.float32),
            pltpu.VMEM((CH,), jnp.float32),
            pltpu.VMEM((CH, H), jnp.float32),
            pltpu.VMEM((CH, H), jnp.float32),
            pltpu.SemaphoreType.DMA,
        ],
    )
    def k(ys_hbm, pos1_hbm, pos2_hbm, w1_hbm, w2_hbm, out_hbm,
          idx1_v, idx2_v, wa_v, wb_v, a_v, b_v, sem):
        wid = lax.axis_index("s") * NC + lax.axis_index("c")
        for sub in range(TPW // CH):
            base = wid * TPW + sub * CH
            pltpu.sync_copy(pos1_hbm.at[pl.ds(base, CH)], idx1_v)
            pltpu.sync_copy(pos2_hbm.at[pl.ds(base, CH)], idx2_v)
            pltpu.sync_copy(w1_hbm.at[pl.ds(base, CH)], wa_v)
            pltpu.sync_copy(w2_hbm.at[pl.ds(base, CH)], wb_v)
            c1 = pltpu.async_copy(ys_hbm.at[idx1_v], a_v, sem)
            c2 = pltpu.async_copy(ys_hbm.at[idx2_v], b_v, sem)
            c1.wait()
            c2.wait()
            war = wa_v[...]
            wbr = wb_v[...]

            def body(i, carry):
                idx = jnp.full((16,), i, jnp.int32)
                wa = war.at[idx].get(mode="promise_in_bounds")
                wb = wbr.at[idx].get(mode="promise_in_bounds")
                for j in range(H // 16):
                    sl = pl.ds(j * 16, 16)
                    a_v[i, sl] = a_v[i, sl] * wa + b_v[i, sl] * wb
                return carry

            lax.fori_loop(0, CH, body, 0)
            pltpu.sync_copy(a_v, out_hbm.at[pl.ds(base, CH)])

    return k(ys, pos1, pos2, w1, w2)


def kernel(hidden_states, gate_w, W1, b1, W2, b2):
    Bq, Lq, Hq = hidden_states.shape
    flat = hidden_states.reshape(Bq * Lq, Hq)
    gwp = jnp.pad(gate_w, ((0, LANES - E), (0, 0)))
    (pos1, pos2, w1r, w2r, eob, nba,
     isf, pfx, slot, haspf) = _route(flat, gwp)
    pos1 = pos1.reshape(T)
    pos2 = pos2.reshape(T)
    w1v = w1r.reshape(T)
    w2v = w2r.reshape(T)
    xs = _dispatch(flat, pos1, pos2)
    ys = _gmm(xs, W1, b1, W2, b2, eob.reshape(NB2), nba.reshape(1),
              isf.reshape(NB2), pfx.reshape(NB2), slot.reshape(NB2),
              haspf.reshape(NB2))
    out = _combine(ys, pos1, pos2, w1v, w2v)
    return out.reshape(Bq, Lq, Hq)


# speedup vs baseline: 1.0434x; 1.0434x over previous
import functools

import jax
import jax.numpy as jnp
from jax import lax
from jax.experimental import pallas as pl
from jax.experimental.pallas import tpu as pltpu
from jax.experimental.pallas import tpu_sc as plsc

T = 2048
H = 1024
F = 2048
E = 8
BS = 128
BS_LOG = 7
NB = 40
NB2 = 64
NP = NB * BS
LANES = 128
NCK = 4
NC, NS = 2, 16
NW = NC * NS
TPW = T // NW


def _route_body(x_ref, g_ref, pos1_ref, pos2_ref, w1_ref, w2_ref,
                eob_ref, nba_ref, isf_ref, pfx_ref, slot_ref, haspf_ref):
    x = x_ref[...]
    g = g_ref[...]
    logits = lax.dot_general(x, g, (((1,), (1,)), ((), ())),
                             preferred_element_type=jnp.float32)
    lane = lax.broadcasted_iota(jnp.int32, (T, LANES), 1)
    logits = jnp.where(lane < E, logits, jnp.float32(-1e30))
    m = jnp.max(logits, axis=1, keepdims=True)
    p = jnp.exp(logits - m)
    p = jnp.where(lane < E, p, 0.0)
    probs = p / jnp.sum(p, axis=1, keepdims=True)
    p1 = jnp.max(probs, axis=1, keepdims=True)
    e1 = jnp.min(jnp.where(probs == p1, lane, LANES), axis=1, keepdims=True)
    probs_m = jnp.where(lane == e1, -1.0, probs)
    p2 = jnp.max(probs_m, axis=1, keepdims=True)
    e2 = jnp.min(jnp.where(probs_m == p2, lane, LANES), axis=1, keepdims=True)
    tot = p1 + p2
    w1_ref[...] = p1 / tot
    w2_ref[...] = p2 / tot
    cnt = (lane == e1).astype(jnp.int32) + (lane == e2).astype(jnp.int32)
    inc = cnt
    sdist = 1
    while sdist < T:
        inc = inc + jnp.concatenate(
            [jnp.zeros((sdist, LANES), jnp.int32), inc[:T - sdist]], axis=0)
        sdist *= 2
    exc = inc - cnt
    n = inc[T - 1:T, :]
    nb = lax.shift_right_logical(n + (BS - 1), BS_LOG)
    spad = lax.shift_left(nb, BS_LOG)
    binc = nb
    ainc = spad
    for sh in (1, 2, 4):
        z = jnp.zeros((1, sh), jnp.int32)
        binc = binc + jnp.concatenate([z, binc[:, :LANES - sh]], axis=1)
        ainc = ainc + jnp.concatenate([z, ainc[:, :LANES - sh]], axis=1)
    aexc = ainc - spad
    posb = exc + aexc
    pos1_ref[...] = jnp.sum(jnp.where(lane == e1, posb, 0), axis=1,
                            keepdims=True)
    pos2_ref[...] = jnp.sum(jnp.where(lane == e2, posb, 0), axis=1,
                            keepdims=True)
    lane1 = lax.broadcasted_iota(jnp.int32, (1, LANES), 1)
    lastact = jnp.max(jnp.where((n > 0) & (lane1 < E), lane1, 0), axis=1,
                      keepdims=True)
    biota = lax.broadcasted_iota(jnp.int32, (NB2, LANES), 0)
    laneb = lax.broadcasted_iota(jnp.int32, (NB2, LANES), 1)
    ge = (biota >= binc) & (laneb < E)
    cntge = jnp.sum(ge.astype(jnp.int32), axis=1, keepdims=True)
    eobv = jnp.minimum(cntge, lastact)
    eob_ref[...] = eobv
    nba_ref[...] = jnp.sum(jnp.where(lane1 < E, nb, 0), axis=1, keepdims=True)
    prev = jnp.concatenate(
        [jnp.full((1, 1), -1, jnp.int32), eobv[:NB2 - 1]], axis=0)
    isf_ref[...] = (eobv != prev).astype(jnp.int32)
    activeb = n > 0
    cand = jnp.where(activeb & (laneb > eobv) & (laneb < E), laneb, LANES)
    nxt = jnp.min(cand, axis=1, keepdims=True)
    haspf_ref[...] = (nxt < E).astype(jnp.int32)
    pfx_ref[...] = jnp.where(nxt < E, nxt, eobv)
    ordv = jnp.sum(jnp.where(activeb & (laneb <= eobv) & (laneb < E), 1, 0),
                   axis=1, keepdims=True)
    slot_ref[...] = jnp.bitwise_and(ordv - 1, 1)


def _route(flat, gwp):
    return pl.pallas_call(
        _route_body,
        out_shape=[
            jax.ShapeDtypeStruct((T, 1), jnp.int32),
            jax.ShapeDtypeStruct((T, 1), jnp.int32),
            jax.ShapeDtypeStruct((T, 1), jnp.float32),
            jax.ShapeDtypeStruct((T, 1), jnp.float32),
            jax.ShapeDtypeStruct((NB2, 1), jnp.int32),
            jax.ShapeDtypeStruct((1, 1), jnp.int32),
            jax.ShapeDtypeStruct((NB2, 1), jnp.int32),
            jax.ShapeDtypeStruct((NB2, 1), jnp.int32),
            jax.ShapeDtypeStruct((NB2, 1), jnp.int32),
            jax.ShapeDtypeStruct((NB2, 1), jnp.int32),
        ],
    )(flat, gwp)


def _dispatch(flat, pos1, pos2):
    mesh = plsc.VectorSubcoreMesh(core_axis_name="c", subcore_axis_name="s")

    @functools.partial(
        pl.kernel,
        mesh=mesh,
        out_type=jax.ShapeDtypeStruct((NP, H), jnp.float32),
        scratch_types=[
            pltpu.VMEM((TPW,), jnp.int32),
            pltpu.VMEM((TPW,), jnp.int32),
            pltpu.VMEM((TPW, H), jnp.float32),
            pltpu.SemaphoreType.DMA,
        ],
    )
    def k(flat_hbm, pos1_hbm, pos2_hbm, xs_hbm, idx1_v, idx2_v, buf_v, sem):
        wid = lax.axis_index("s") * NC + lax.axis_index("c")
        base = wid * TPW
        pltpu.sync_copy(pos1_hbm.at[pl.ds(base, TPW)], idx1_v)
        pltpu.sync_copy(pos2_hbm.at[pl.ds(base, TPW)], idx2_v)
        pltpu.sync_copy(flat_hbm.at[pl.ds(base, TPW)], buf_v)
        c1 = pltpu.async_copy(buf_v, xs_hbm.at[idx1_v], sem)
        c2 = pltpu.async_copy(buf_v, xs_hbm.at[idx2_v], sem)
        c1.wait()
        c2.wait()

    return k(flat, pos1, pos2)


def _gmm_body(eob_ref, nba_ref, isf_ref, pfx_ref, slot_ref, haspf_ref,
              x_ref, w1_any, b1_ref, w2_any, b2_ref, o_ref,
              w1buf, w2buf, sem):
    b = pl.program_id(0)
    s = slot_ref[b]

    def _start_weight_copy(e, slot):
        for c in range(NCK):
            pltpu.make_async_copy(
                w1_any.at[e, pl.ds(c * (F // NCK), F // NCK)],
                w1buf.at[slot, pl.ds(c * (F // NCK), F // NCK)],
                sem.at[slot]).start()
            pltpu.make_async_copy(
                w2_any.at[e, pl.ds(c * (H // NCK), H // NCK)],
                w2buf.at[slot, pl.ds(c * (H // NCK), H // NCK)],
                sem.at[slot]).start()

    @pl.when(b == 0)
    def _():
        _start_weight_copy(eob_ref[0], 0)

    @pl.when((isf_ref[b] == 1) & (haspf_ref[b] == 1))
    def _():
        _start_weight_copy(pfx_ref[b], 1 - s)

    @pl.when(isf_ref[b] == 1)
    def _():
        for c in range(NCK):
            pltpu.make_async_copy(
                w1_any.at[0, pl.ds(0, F // NCK)],
                w1buf.at[s, pl.ds(0, F // NCK)], sem.at[s]).wait()
            pltpu.make_async_copy(
                w2_any.at[0, pl.ds(0, H // NCK)],
                w2buf.at[s, pl.ds(0, H // NCK)], sem.at[s]).wait()

    @pl.when(b < nba_ref[0])
    def _():
        x = x_ref[...]
        h = lax.dot_general(x, w1buf[s], (((1,), (1,)), ((), ())),
                            preferred_element_type=jnp.float32)
        h = jnp.maximum(h + b1_ref[...], 0.0)
        y = lax.dot_general(h, w2buf[s], (((1,), (1,)), ((), ())),
                            preferred_element_type=jnp.float32)
        o_ref[...] = y + b2_ref[...]


def _gmm(xs, W1, b1, W2, b2, eob, nba, isf, pfx, slot, haspf):
    grid_spec = pltpu.PrefetchScalarGridSpec(
        num_scalar_prefetch=6,
        grid=(NB,),
        in_specs=[
            pl.BlockSpec((BS, H), lambda b, *sp: (b, 0)),
            pl.BlockSpec(memory_space=pl.ANY),
            pl.BlockSpec((None, 1, F), lambda b, *sp: (sp[0][b], 0, 0)),
            pl.BlockSpec(memory_space=pl.ANY),
            pl.BlockSpec((None, 1, H), lambda b, *sp: (sp[0][b], 0, 0)),
        ],
        out_specs=pl.BlockSpec((BS, H), lambda b, *sp: (b, 0)),
        scratch_shapes=[
            pltpu.VMEM((2, F, H), jnp.float32),
            pltpu.VMEM((2, H, F), jnp.float32),
            pltpu.SemaphoreType.DMA((2,)),
        ],
    )
    return pl.pallas_call(
        _gmm_body,
        grid_spec=grid_spec,
        out_shape=jax.ShapeDtypeStruct((NP, H), jnp.float32),
        compiler_params=pltpu.CompilerParams(
            dimension_semantics=("arbitrary",)),
    )(eob, nba, isf, pfx, slot, haspf,
      xs, W1, b1.reshape(E, 1, F), W2, b2.reshape(E, 1, H))


def _combine(ys, pos1, pos2, w1, w2):
    mesh = plsc.VectorSubcoreMesh(core_axis_name="c", subcore_axis_name="s")
    CH = 16
    NSUB = TPW // CH

    @functools.partial(
        pl.kernel,
        mesh=mesh,
        out_type=jax.ShapeDtypeStruct((T, H), jnp.float32),
        scratch_types=[
            pltpu.VMEM((TPW,), jnp.int32),
            pltpu.VMEM((TPW,), jnp.int32),
            pltpu.VMEM((TPW,), jnp.float32),
            pltpu.VMEM((TPW,), jnp.float32),
            pltpu.VMEM((2, CH, H), jnp.float32),
            pltpu.VMEM((2, CH, H), jnp.float32),
            pltpu.VMEM((2, CH, H), jnp.float32),
            pltpu.SemaphoreType.DMA,
            pltpu.SemaphoreType.DMA,
        ],
    )
    def k(ys_hbm, pos1_hbm, pos2_hbm, w1_hbm, w2_hbm, out_hbm,
          idx1_v, idx2_v, wa_v, wb_v, a_v, b_v, o_v, sem, sem_o):
        wid = lax.axis_index("s") * NC + lax.axis_index("c")
        base = wid * TPW
        pltpu.sync_copy(pos1_hbm.at[pl.ds(base, TPW)], idx1_v)
        pltpu.sync_copy(pos2_hbm.at[pl.ds(base, TPW)], idx2_v)
        pltpu.sync_copy(w1_hbm.at[pl.ds(base, TPW)], wa_v)
        pltpu.sync_copy(w2_hbm.at[pl.ds(base, TPW)], wb_v)

        def issue(sub, p):
            sl = pl.ds(sub * CH, CH)
            c1 = pltpu.async_copy(ys_hbm.at[idx1_v.at[sl]], a_v.at[p], sem)
            c2 = pltpu.async_copy(ys_hbm.at[idx2_v.at[sl]], b_v.at[p], sem)
            return c1, c2

        gath = {0: issue(0, 0)}
        stores = {}
        for sub in range(NSUB):
            p = sub % 2
            c1, c2 = gath.pop(sub)
            c1.wait()
            c2.wait()
            if sub + 1 < NSUB:
                gath[sub + 1] = issue(sub + 1, 1 - p)
            if sub >= 2:
                stores.pop(sub - 2).wait()
            war = wa_v[pl.ds(sub * CH, CH)]
            wbr = wb_v[pl.ds(sub * CH, CH)]

            def body(i, carry):
                idx = jnp.full((16,), i, jnp.int32)
                wa = war.at[idx].get(mode="promise_in_bounds")
                wb = wbr.at[idx].get(mode="promise_in_bounds")
                for j in range(H // 16):
                    jsl = pl.ds(j * 16, 16)
                    o_v[p, i, jsl] = a_v[p, i, jsl] * wa + b_v[p, i, jsl] * wb
                return carry

            lax.fori_loop(0, CH, body, 0)
            stores[sub] = pltpu.async_copy(
                o_v.at[p], out_hbm.at[pl.ds(base + sub * CH, CH)], sem_o)
        for sub in sorted(stores):
            stores.pop(sub).wait()

    return k(ys, pos1, pos2, w1, w2)


def kernel(hidden_states, gate_w, W1, b1, W2, b2):
    Bq, Lq, Hq = hidden_states.shape
    flat = hidden_states.reshape(Bq * Lq, Hq)
    gwp = jnp.pad(gate_w, ((0, LANES - E), (0, 0)))
    (pos1, pos2, w1r, w2r, eob, nba,
     isf, pfx, slot, haspf) = _route(flat, gwp)
    pos1 = pos1.reshape(T)
    pos2 = pos2.reshape(T)
    w1v = w1r.reshape(T)
    w2v = w2r.reshape(T)
    xs = _dispatch(flat, pos1, pos2)
    ys = _gmm(xs, W1, b1, W2, b2, eob.reshape(NB2), nba.reshape(1),
              isf.reshape(NB2), pfx.reshape(NB2), slot.reshape(NB2),
              haspf.reshape(NB2))
    out = _combine(ys, pos1, pos2, w1v, w2v)
    return out.reshape(Bq, Lq, Hq)


# speedup vs baseline: 1.0588x; 1.0148x over previous
import functools

import jax
import jax.numpy as jnp
from jax import lax
from jax.experimental import pallas as pl
from jax.experimental.pallas import tpu as pltpu
from jax.experimental.pallas import tpu_sc as plsc

T = 2048
H = 1024
F = 2048
E = 8
BS = 128
BS_LOG = 7
NB = 40
NB2 = 64
NP = NB * BS
LANES = 128
NCK = 4
NC, NS = 2, 16
NW = NC * NS
TPW = T // NW


def _route_body(x_ref, g_ref, pos1_ref, pos2_ref, w1_ref, w2_ref,
                eob_ref, nba_ref, isf_ref, pfx_ref, slot_ref, haspf_ref):
    x = x_ref[...]
    g = g_ref[...]
    logits = lax.dot_general(x, g, (((1,), (1,)), ((), ())),
                             preferred_element_type=jnp.float32)
    lane = lax.broadcasted_iota(jnp.int32, (T, LANES), 1)
    logits = jnp.where(lane < E, logits, jnp.float32(-1e30))
    m = jnp.max(logits, axis=1, keepdims=True)
    p = jnp.exp(logits - m)
    p = jnp.where(lane < E, p, 0.0)
    probs = p / jnp.sum(p, axis=1, keepdims=True)
    p1 = jnp.max(probs, axis=1, keepdims=True)
    e1 = jnp.min(jnp.where(probs == p1, lane, LANES), axis=1, keepdims=True)
    probs_m = jnp.where(lane == e1, -1.0, probs)
    p2 = jnp.max(probs_m, axis=1, keepdims=True)
    e2 = jnp.min(jnp.where(probs_m == p2, lane, LANES), axis=1, keepdims=True)
    tot = p1 + p2
    w1_ref[...] = p1 / tot
    w2_ref[...] = p2 / tot
    cnt = (lane == e1).astype(jnp.int32) + (lane == e2).astype(jnp.int32)
    inc = cnt
    sdist = 1
    while sdist < T:
        inc = inc + jnp.concatenate(
            [jnp.zeros((sdist, LANES), jnp.int32), inc[:T - sdist]], axis=0)
        sdist *= 2
    exc = inc - cnt
    n = inc[T - 1:T, :]
    nb = lax.shift_right_logical(n + (BS - 1), BS_LOG)
    spad = lax.shift_left(nb, BS_LOG)
    binc = nb
    ainc = spad
    for sh in (1, 2, 4):
        z = jnp.zeros((1, sh), jnp.int32)
        binc = binc + jnp.concatenate([z, binc[:, :LANES - sh]], axis=1)
        ainc = ainc + jnp.concatenate([z, ainc[:, :LANES - sh]], axis=1)
    aexc = ainc - spad
    posb = exc + aexc
    pos1_ref[...] = jnp.sum(jnp.where(lane == e1, posb, 0), axis=1,
                            keepdims=True)
    pos2_ref[...] = jnp.sum(jnp.where(lane == e2, posb, 0), axis=1,
                            keepdims=True)
    lane1 = lax.broadcasted_iota(jnp.int32, (1, LANES), 1)
    lastact = jnp.max(jnp.where((n > 0) & (lane1 < E), lane1, 0), axis=1,
                      keepdims=True)
    biota = lax.broadcasted_iota(jnp.int32, (NB2, LANES), 0)
    laneb = lax.broadcasted_iota(jnp.int32, (NB2, LANES), 1)
    ge = (biota >= binc) & (laneb < E)
    cntge = jnp.sum(ge.astype(jnp.int32), axis=1, keepdims=True)
    eobv = jnp.minimum(cntge, lastact)
    eob_ref[...] = eobv
    nba_ref[...] = jnp.sum(jnp.where(lane1 < E, nb, 0), axis=1, keepdims=True)
    prev = jnp.concatenate(
        [jnp.full((1, 1), -1, jnp.int32), eobv[:NB2 - 1]], axis=0)
    isf_ref[...] = (eobv != prev).astype(jnp.int32)
    activeb = n > 0
    cand = jnp.where(activeb & (laneb > eobv) & (laneb < E), laneb, LANES)
    nxt = jnp.min(cand, axis=1, keepdims=True)
    haspf_ref[...] = (nxt < E).astype(jnp.int32)
    pfx_ref[...] = jnp.where(nxt < E, nxt, eobv)
    ordv = jnp.sum(jnp.where(activeb & (laneb <= eobv) & (laneb < E), 1, 0),
                   axis=1, keepdims=True)
    slot_ref[...] = jnp.bitwise_and(ordv - 1, 1)


def _route(flat, gwp):
    return pl.pallas_call(
        _route_body,
        out_shape=[
            jax.ShapeDtypeStruct((T, 1), jnp.int32),
            jax.ShapeDtypeStruct((T, 1), jnp.int32),
            jax.ShapeDtypeStruct((T, 1), jnp.float32),
            jax.ShapeDtypeStruct((T, 1), jnp.float32),
            jax.ShapeDtypeStruct((NB2, 1), jnp.int32),
            jax.ShapeDtypeStruct((1, 1), jnp.int32),
            jax.ShapeDtypeStruct((NB2, 1), jnp.int32),
            jax.ShapeDtypeStruct((NB2, 1), jnp.int32),
            jax.ShapeDtypeStruct((NB2, 1), jnp.int32),
            jax.ShapeDtypeStruct((NB2, 1), jnp.int32),
        ],
    )(flat, gwp)


def _dispatch(flat, pos1, pos2):
    mesh = plsc.VectorSubcoreMesh(core_axis_name="c", subcore_axis_name="s")

    @functools.partial(
        pl.kernel,
        mesh=mesh,
        out_type=jax.ShapeDtypeStruct((NP, H), jnp.float32),
        scratch_types=[
            pltpu.VMEM((TPW,), jnp.int32),
            pltpu.VMEM((TPW,), jnp.int32),
            pltpu.VMEM((TPW, H), jnp.float32),
            pltpu.SemaphoreType.DMA,
        ],
    )
    def k(flat_hbm, pos1_hbm, pos2_hbm, xs_hbm, idx1_v, idx2_v, buf_v, sem):
        wid = lax.axis_index("s") * NC + lax.axis_index("c")
        base = wid * TPW
        l1 = pltpu.async_copy(pos1_hbm.at[pl.ds(base, TPW)], idx1_v, sem)
        l2 = pltpu.async_copy(pos2_hbm.at[pl.ds(base, TPW)], idx2_v, sem)
        l3 = pltpu.async_copy(flat_hbm.at[pl.ds(base, TPW)], buf_v, sem)
        l1.wait()
        l2.wait()
        l3.wait()
        c1 = pltpu.async_copy(buf_v, xs_hbm.at[idx1_v], sem)
        c2 = pltpu.async_copy(buf_v, xs_hbm.at[idx2_v], sem)
        c1.wait()
        c2.wait()

    return k(flat, pos1, pos2)


def _gmm_body(eob_ref, nba_ref, isf_ref, pfx_ref, slot_ref, haspf_ref,
              x_ref, w1_any, b1_ref, w2_any, b2_ref, o_ref,
              w1buf, w2buf, sem):
    b = pl.program_id(0)
    s = slot_ref[b]

    def _start_weight_copy(e, slot):
        for c in range(NCK):
            pltpu.make_async_copy(
                w1_any.at[e, pl.ds(c * (F // NCK), F // NCK)],
                w1buf.at[slot, pl.ds(c * (F // NCK), F // NCK)],
                sem.at[slot]).start()
            pltpu.make_async_copy(
                w2_any.at[e, pl.ds(c * (H // NCK), H // NCK)],
                w2buf.at[slot, pl.ds(c * (H // NCK), H // NCK)],
                sem.at[slot]).start()

    @pl.when(b == 0)
    def _():
        _start_weight_copy(eob_ref[0], 0)

    @pl.when((isf_ref[b] == 1) & (haspf_ref[b] == 1))
    def _():
        _start_weight_copy(pfx_ref[b], 1 - s)

    @pl.when(isf_ref[b] == 1)
    def _():
        for c in range(NCK):
            pltpu.make_async_copy(
                w1_any.at[0, pl.ds(0, F // NCK)],
                w1buf.at[s, pl.ds(0, F // NCK)], sem.at[s]).wait()
            pltpu.make_async_copy(
                w2_any.at[0, pl.ds(0, H // NCK)],
                w2buf.at[s, pl.ds(0, H // NCK)], sem.at[s]).wait()

    @pl.when(b < nba_ref[0])
    def _():
        x = x_ref[...]
        h = lax.dot_general(x, w1buf[s], (((1,), (1,)), ((), ())),
                            preferred_element_type=jnp.float32)
        h = jnp.maximum(h + b1_ref[...], 0.0)
        y = lax.dot_general(h, w2buf[s], (((1,), (1,)), ((), ())),
                            preferred_element_type=jnp.float32)
        o_ref[...] = y + b2_ref[...]


def _gmm(xs, W1, b1, W2, b2, eob, nba, isf, pfx, slot, haspf):
    grid_spec = pltpu.PrefetchScalarGridSpec(
        num_scalar_prefetch=6,
        grid=(NB,),
        in_specs=[
            pl.BlockSpec((BS, H), lambda b, *sp: (b, 0)),
            pl.BlockSpec(memory_space=pl.ANY),
            pl.BlockSpec((None, 1, F), lambda b, *sp: (sp[0][b], 0, 0)),
            pl.BlockSpec(memory_space=pl.ANY),
            pl.BlockSpec((None, 1, H), lambda b, *sp: (sp[0][b], 0, 0)),
        ],
        out_specs=pl.BlockSpec((BS, H), lambda b, *sp: (b, 0)),
        scratch_shapes=[
            pltpu.VMEM((2, F, H), jnp.float32),
            pltpu.VMEM((2, H, F), jnp.float32),
            pltpu.SemaphoreType.DMA((2,)),
        ],
    )
    return pl.pallas_call(
        _gmm_body,
        grid_spec=grid_spec,
        out_shape=jax.ShapeDtypeStruct((NP, H), jnp.float32),
        compiler_params=pltpu.CompilerParams(
            dimension_semantics=("arbitrary",)),
    )(eob, nba, isf, pfx, slot, haspf,
      xs, W1, b1.reshape(E, 1, F), W2, b2.reshape(E, 1, H))


def _combine(ys, pos1, pos2, w1, w2):
    mesh = plsc.VectorSubcoreMesh(core_axis_name="c", subcore_axis_name="s")
    CH = 16
    NSUB = TPW // CH

    @functools.partial(
        pl.kernel,
        mesh=mesh,
        out_type=jax.ShapeDtypeStruct((T, H), jnp.float32),
        scratch_types=[
            pltpu.VMEM((TPW,), jnp.int32),
            pltpu.VMEM((TPW,), jnp.int32),
            pltpu.VMEM((TPW,), jnp.float32),
            pltpu.VMEM((TPW,), jnp.float32),
            pltpu.VMEM((2, CH, H), jnp.float32),
            pltpu.VMEM((2, CH, H), jnp.float32),
            pltpu.VMEM((2, CH, H), jnp.float32),
            pltpu.SemaphoreType.DMA,
            pltpu.SemaphoreType.DMA,
        ],
    )
    def k(ys_hbm, pos1_hbm, pos2_hbm, w1_hbm, w2_hbm, out_hbm,
          idx1_v, idx2_v, wa_v, wb_v, a_v, b_v, o_v, sem, sem_o):
        wid = lax.axis_index("s") * NC + lax.axis_index("c")
        base = wid * TPW
        l1 = pltpu.async_copy(pos1_hbm.at[pl.ds(base, TPW)], idx1_v, sem)
        l2 = pltpu.async_copy(pos2_hbm.at[pl.ds(base, TPW)], idx2_v, sem)
        l3 = pltpu.async_copy(w1_hbm.at[pl.ds(base, TPW)], wa_v, sem)
        l4 = pltpu.async_copy(w2_hbm.at[pl.ds(base, TPW)], wb_v, sem)
        l1.wait()
        l2.wait()
        l3.wait()
        l4.wait()

        def issue(sub, p):
            sl = pl.ds(sub * CH, CH)
            c1 = pltpu.async_copy(ys_hbm.at[idx1_v.at[sl]], a_v.at[p], sem)
            c2 = pltpu.async_copy(ys_hbm.at[idx2_v.at[sl]], b_v.at[p], sem)
            return c1, c2

        gath = {0: issue(0, 0)}
        stores = {}
        for sub in range(NSUB):
            p = sub % 2
            c1, c2 = gath.pop(sub)
            c1.wait()
            c2.wait()
            if sub + 1 < NSUB:
                gath[sub + 1] = issue(sub + 1, 1 - p)
            if sub >= 2:
                stores.pop(sub - 2).wait()
            war = wa_v[pl.ds(sub * CH, CH)]
            wbr = wb_v[pl.ds(sub * CH, CH)]

            def body(i, carry):
                idx = jnp.full((16,), i, jnp.int32)
                wa = war.at[idx].get(mode="promise_in_bounds")
                wb = wbr.at[idx].get(mode="promise_in_bounds")
                for j in range(H // 16):
                    jsl = pl.ds(j * 16, 16)
                    o_v[p, i, jsl] = a_v[p, i, jsl] * wa + b_v[p, i, jsl] * wb
                return carry

            lax.fori_loop(0, CH, body, 0)
            stores[sub] = pltpu.async_copy(
                o_v.at[p], out_hbm.at[pl.ds(base + sub * CH, CH)], sem_o)
        for sub in sorted(stores):
            stores.pop(sub).wait()

    return k(ys, pos1, pos2, w1, w2)


def kernel(hidden_states, gate_w, W1, b1, W2, b2):
    Bq, Lq, Hq = hidden_states.shape
    flat = hidden_states.reshape(Bq * Lq, Hq)
    gwp = jnp.pad(gate_w, ((0, LANES - E), (0, 0)))
    (pos1, pos2, w1r, w2r, eob, nba,
     isf, pfx, slot, haspf) = _route(flat, gwp)
    pos1 = pos1.reshape(T)
    pos2 = pos2.reshape(T)
    w1v = w1r.reshape(T)
    w2v = w2r.reshape(T)
    xs = _dispatch(flat, pos1, pos2)
    ys = _gmm(xs, W1, b1, W2, b2, eob.reshape(NB2), nba.reshape(1),
              isf.reshape(NB2), pfx.reshape(NB2), slot.reshape(NB2),
              haspf.reshape(NB2))
    out = _combine(ys, pos1, pos2, w1v, w2v)
    return out.reshape(Bq, Lq, Hq)
